# baseline (speedup 1.0000x reference)
import functools

import jax
import jax.numpy as jnp
from jax import lax
from jax.experimental import pallas as pl
from jax.experimental.pallas import tpu as pltpu
from jax.experimental.pallas import tpu_sc as plsc

NC = 2
NS = 16
NW = NC * NS
CHUNK = 64
NBUF = 5


def _aggregate_sc(x, src1, dst1, n_chunks, n_pad):
    n, d = x.shape
    epw = n_chunks * CHUNK
    rows_per_tile = n_pad // NS

    mesh = plsc.VectorSubcoreMesh(
        core_axis_name="c", subcore_axis_name="s", num_cores=NC,
        num_subcores=NS)

    @functools.partial(
        pl.kernel,
        out_type=(
            jax.ShapeDtypeStruct((NC, n_pad, d), jnp.float32),
            jax.ShapeDtypeStruct((NC * n_pad,), jnp.float32),
        ),
        mesh=mesh,
        scratch_types=[
            pltpu.VMEM((NBUF, CHUNK), jnp.int32),
            pltpu.VMEM((NBUF, CHUNK), jnp.int32),
            pltpu.VMEM((NBUF, CHUNK, d), jnp.float32),
            pltpu.VMEM((CHUNK,), jnp.float32),
            pltpu.VMEM((640,), jnp.float32),
            pltpu.VMEM_SHARED((n_pad, d), jnp.float32),
            pltpu.VMEM_SHARED((n_pad,), jnp.float32),
            pltpu.SemaphoreType.DMA((NBUF,)),
            pltpu.SemaphoreType.DMA((NBUF,)),
        ],
    )
    def agg(x_hbm, src_hbm, dst_hbm, zrows_hbm,
            nei_out, deg_out,
            src_ring, dst_ring, rows, ones_v, dbuf, acc, dacc, gsem, isem):
        cid = lax.axis_index("c")
        sid = lax.axis_index("s")
        wid = sid * NC + cid
        dpt = rows_per_tile

        zsl = pl.ds(sid * rows_per_tile, rows_per_tile)
        pltpu.sync_copy(zrows_hbm, acc.at[zsl])
        for j in range(640 // 16):
            dbuf[pl.ds(j * 16, 16)] = jnp.zeros((16,), jnp.float32)
        pltpu.sync_copy(dbuf.at[pl.ds(0, dpt)], dacc.at[pl.ds(sid * dpt, dpt)])
        for j in range(CHUNK // 16):
            ones_v[pl.ds(j * 16, 16)] = jnp.ones((16,), jnp.float32)
        plsc.subcore_barrier()

        def load_idx(c, slot, sem_wait=False):
            base = pl.multiple_of(wid * epw + c * CHUNK, 8)
            s = pltpu.async_copy(
                src_hbm.at[pl.ds(base, CHUNK)], src_ring.at[slot],
                isem.at[slot])
            t = pltpu.async_copy(
                dst_hbm.at[pl.ds(base, CHUNK)], dst_ring.at[slot],
                isem.at[slot])
            if sem_wait:
                s.wait()
                t.wait()

        def wait_idx(slot):
            pltpu.make_async_copy(
                src_hbm.at[pl.ds(0, CHUNK)], src_ring.at[slot],
                isem.at[slot]).wait()
            pltpu.make_async_copy(
                dst_hbm.at[pl.ds(0, CHUNK)], dst_ring.at[slot],
                isem.at[slot]).wait()

        def gather(slot):
            pltpu.async_copy(
                x_hbm.at[src_ring.at[slot]], rows.at[slot], gsem.at[slot])

        for k in range(NBUF):
            load_idx(k, k)
        for k in range(NBUF - 1):
            wait_idx(k)
            gather(k)

        def body(i, carry):
            b = lax.rem(i, NBUF)
            pltpu.make_async_copy(
                x_hbm.at[src_ring.at[0]], rows.at[b], gsem.at[b]).wait()
            pltpu.sync_copy(rows.at[b], acc.at[dst_ring.at[b]], add=True)
            pltpu.sync_copy(ones_v, dacc.at[dst_ring.at[b]], add=True)
            nl = i + NBUF

            @pl.when(nl < n_chunks)
            def _load():
                load_idx(nl, b)

            c = i + NBUF - 1
            bc = lax.rem(c, NBUF)

            @pl.when(c < n_chunks)
            def _gather():
                wait_idx(bc)
                gather(bc)
            return carry

        lax.fori_loop(0, n_chunks, body, 0)
        plsc.subcore_barrier()

        pltpu.sync_copy(acc.at[zsl], nei_out.at[cid, zsl])
        pltpu.sync_copy(dacc.at[pl.ds(sid * dpt, dpt)], dbuf.at[pl.ds(0, dpt)])
        dsl = pl.ds(pl.multiple_of(cid * n_pad + sid * dpt, 8), dpt)
        pltpu.sync_copy(dbuf.at[pl.ds(0, dpt)], deg_out.at[dsl])

    zrows = jnp.zeros((rows_per_tile, d), jnp.float32)
    return agg(x, src1, dst1, zrows)


def _linear_tc(x, nei, deg, w, b, row_block):
    n, d = x.shape

    def body(x_ref, n_ref, d_ref, w_ref, b_ref, o_ref):
        s = x_ref[...] + n_ref[0] + n_ref[1]
        dv = d_ref[0] + d_ref[1] + 1.0
        inv = (1.0 / dv)[:, None]
        s = s * inv
        o_ref[...] = lax.dot_general(
            s, w_ref[...], (((1,), (1,)), ((), ())),
            preferred_element_type=jnp.float32,
            precision=lax.Precision.HIGHEST) + b_ref[...]

    grid = (pl.cdiv(n, row_block),)
    return pl.pallas_call(
        body,
        grid=grid,
        in_specs=[
            pl.BlockSpec((row_block, d), lambda i: (i, 0)),
            pl.BlockSpec((NC, row_block, d), lambda i: (0, i, 0)),
            pl.BlockSpec((NC, row_block), lambda i: (0, i)),
            pl.BlockSpec((d, d), lambda i: (0, 0)),
            pl.BlockSpec((1, d), lambda i: (0, 0)),
        ],
        out_specs=pl.BlockSpec((row_block, d), lambda i: (i, 0)),
        out_shape=jax.ShapeDtypeStruct((n, d), jnp.float32),
    )(x, nei, deg, w, b)


def kernel(x, edge_index, W, b):
    n, d = x.shape
    e = edge_index.shape[1]
    src = edge_index[0]
    dst = edge_index[1]

    n_pad = ((n + 8 * NS - 1) // (8 * NS)) * (8 * NS)
    epw = e // NW
    epw_pad = ((epw + CHUNK - 1) // CHUNK) * CHUNK
    padn = epw_pad - epw

    srcw = src.reshape(NW, epw)
    dstw = dst.reshape(NW, epw)
    if padn:
        ps = (jnp.arange(NW, dtype=jnp.int32)[:, None] * 131
              + jnp.arange(padn, dtype=jnp.int32)[None, :] * 7) % n
        pd = jnp.broadcast_to(
            n + jnp.arange(NW, dtype=jnp.int32)[:, None], (NW, padn))
        srcw = jnp.concatenate([srcw, ps], axis=1)
        dstw = jnp.concatenate([dstw, pd.astype(jnp.int32)], axis=1)
    src1 = srcw.reshape(NW * epw_pad)
    dst1 = dstw.reshape(NW * epw_pad)

    nei, deg = _aggregate_sc(x, src1, dst1, epw_pad // CHUNK, n_pad)

    deg = deg.reshape(NC, n_pad)
    return _linear_tc(x, nei, deg, W, b.reshape(1, d), row_block=1024)


# speedup vs baseline: 1.0420x; 1.0420x over previous
import functools

import jax
import jax.numpy as jnp
from jax import lax
from jax.experimental import pallas as pl
from jax.experimental.pallas import tpu as pltpu
from jax.experimental.pallas import tpu_sc as plsc

NC = 2
NS = 16
NW = NC * NS
CHUNK = 80
NBUF = 4


def _aggregate_sc(x, src1, dst1, n_chunks, n_pad):
    n, d = x.shape
    epw = n_chunks * CHUNK
    rows_per_tile = n_pad // NS

    mesh = plsc.VectorSubcoreMesh(
        core_axis_name="c", subcore_axis_name="s", num_cores=NC,
        num_subcores=NS)

    @functools.partial(
        pl.kernel,
        out_type=(
            jax.ShapeDtypeStruct((NC, n_pad, d), jnp.float32),
            jax.ShapeDtypeStruct((NC * n_pad,), jnp.float32),
        ),
        mesh=mesh,
        scratch_types=[
            pltpu.VMEM((NBUF, CHUNK), jnp.int32),
            pltpu.VMEM((NBUF, CHUNK), jnp.int32),
            pltpu.VMEM((NBUF, CHUNK, d), jnp.float32),
            pltpu.VMEM((CHUNK,), jnp.float32),
            pltpu.VMEM((640,), jnp.float32),
            pltpu.VMEM_SHARED((n_pad, d), jnp.float32),
            pltpu.VMEM_SHARED((n_pad,), jnp.float32),
            pltpu.SemaphoreType.DMA((NBUF,)),
            pltpu.SemaphoreType.DMA((NBUF,)),
        ],
    )
    def agg(x_hbm, src_hbm, dst_hbm, zrows_hbm,
            nei_out, deg_out,
            src_ring, dst_ring, rows, ones_v, dbuf, acc, dacc, gsem, isem):
        cid = lax.axis_index("c")
        sid = lax.axis_index("s")
        wid = sid * NC + cid
        dpt = rows_per_tile

        zsl = pl.ds(sid * rows_per_tile, rows_per_tile)
        pltpu.sync_copy(zrows_hbm, acc.at[zsl])
        for j in range(640 // 16):
            dbuf[pl.ds(j * 16, 16)] = jnp.zeros((16,), jnp.float32)
        pltpu.sync_copy(dbuf.at[pl.ds(0, dpt)], dacc.at[pl.ds(sid * dpt, dpt)])
        for j in range(CHUNK // 16):
            ones_v[pl.ds(j * 16, 16)] = jnp.ones((16,), jnp.float32)
        plsc.subcore_barrier()

        def load_idx(c, slot, sem_wait=False):
            base = pl.multiple_of(wid * epw + c * CHUNK, 8)
            s = pltpu.async_copy(
                src_hbm.at[pl.ds(base, CHUNK)], src_ring.at[slot],
                isem.at[slot])
            t = pltpu.async_copy(
                dst_hbm.at[pl.ds(base, CHUNK)], dst_ring.at[slot],
                isem.at[slot])
            if sem_wait:
                s.wait()
                t.wait()

        def wait_idx(slot):
            pltpu.make_async_copy(
                src_hbm.at[pl.ds(0, CHUNK)], src_ring.at[slot],
                isem.at[slot]).wait()
            pltpu.make_async_copy(
                dst_hbm.at[pl.ds(0, CHUNK)], dst_ring.at[slot],
                isem.at[slot]).wait()

        def gather(slot):
            pltpu.async_copy(
                x_hbm.at[src_ring.at[slot]], rows.at[slot], gsem.at[slot])

        for k in range(NBUF):
            load_idx(k, k)
        for k in range(NBUF - 1):
            wait_idx(k)
            gather(k)

        def body(i, carry):
            b = lax.rem(i, NBUF)
            pltpu.make_async_copy(
                x_hbm.at[src_ring.at[0]], rows.at[b], gsem.at[b]).wait()
            pltpu.sync_copy(rows.at[b], acc.at[dst_ring.at[b]], add=True)
            pltpu.sync_copy(ones_v, dacc.at[dst_ring.at[b]], add=True)
            nl = i + NBUF

            @pl.when(nl < n_chunks)
            def _load():
                load_idx(nl, b)

            c = i + NBUF - 1
            bc = lax.rem(c, NBUF)

            @pl.when(c < n_chunks)
            def _gather():
                wait_idx(bc)
                gather(bc)
            return carry

        lax.fori_loop(0, n_chunks, body, 0)
        plsc.subcore_barrier()

        pltpu.sync_copy(acc.at[zsl], nei_out.at[cid, zsl])
        pltpu.sync_copy(dacc.at[pl.ds(sid * dpt, dpt)], dbuf.at[pl.ds(0, dpt)])
        dsl = pl.ds(pl.multiple_of(cid * n_pad + sid * dpt, 8), dpt)
        pltpu.sync_copy(dbuf.at[pl.ds(0, dpt)], deg_out.at[dsl])

    zrows = jnp.zeros((rows_per_tile, d), jnp.float32)
    return agg(x, src1, dst1, zrows)


def _linear_tc(x, nei, deg, w, b, row_block):
    n, d = x.shape

    n_pad = nei.shape[1]
    nblk = n_pad // row_block

    def body(x_ref, n_ref, d0_ref, d1_ref, w_ref, b_ref, o_ref):
        s = x_ref[...] + n_ref[0] + n_ref[1]
        dv = d0_ref[...] + d1_ref[...] + 1.0
        inv = (1.0 / dv)[:, None]
        s = s * inv
        o_ref[...] = lax.dot_general(
            s, w_ref[...], (((1,), (1,)), ((), ())),
            preferred_element_type=jnp.float32,
            precision=lax.Precision.HIGHEST) + b_ref[...]

    grid = (pl.cdiv(n, row_block),)
    return pl.pallas_call(
        body,
        grid=grid,
        in_specs=[
            pl.BlockSpec((row_block, d), lambda i: (i, 0)),
            pl.BlockSpec((NC, row_block, d), lambda i: (0, i, 0)),
            pl.BlockSpec((row_block,), lambda i: (i,)),
            pl.BlockSpec((row_block,), lambda i: (i + nblk,)),
            pl.BlockSpec((d, d), lambda i: (0, 0)),
            pl.BlockSpec((1, d), lambda i: (0, 0)),
        ],
        out_specs=pl.BlockSpec((row_block, d), lambda i: (i, 0)),
        out_shape=jax.ShapeDtypeStruct((n, d), jnp.float32),
    )(x, nei, deg, deg, w, b)


def kernel(x, edge_index, W, b):
    n, d = x.shape
    e = edge_index.shape[1]
    src = edge_index[0]
    dst = edge_index[1]

    rb = 1024
    n_pad = ((n + rb - 1) // rb) * rb
    epw = e // NW
    epw_pad = ((epw + CHUNK - 1) // CHUNK) * CHUNK
    padn = epw_pad - epw

    srcw = src.reshape(NW, epw)
    dstw = dst.reshape(NW, epw)
    if padn:
        ps = (jnp.arange(NW, dtype=jnp.int32)[:, None] * 131
              + jnp.arange(padn, dtype=jnp.int32)[None, :] * 7) % n
        pd = jnp.broadcast_to(
            n + jnp.arange(NW, dtype=jnp.int32)[:, None], (NW, padn))
        srcw = jnp.concatenate([srcw, ps], axis=1)
        dstw = jnp.concatenate([dstw, pd.astype(jnp.int32)], axis=1)
    src1 = srcw.reshape(NW * epw_pad)
    dst1 = dstw.reshape(NW * epw_pad)

    nei, deg = _aggregate_sc(x, src1, dst1, epw_pad // CHUNK, n_pad)

    return _linear_tc(x, nei, deg, W, b.reshape(1, d), row_block=rb)


# speedup vs baseline: 1.0600x; 1.0172x over previous
import functools

import jax
import jax.numpy as jnp
from jax import lax
from jax.experimental import pallas as pl
from jax.experimental.pallas import tpu as pltpu
from jax.experimental.pallas import tpu_sc as plsc

NC = 2
NS = 16
NW = NC * NS
CHUNK = 80
NBUF = 4


def _aggregate_sc(x, src1, dst1, n_chunks, n_pad):
    n, d = x.shape
    epw = n_chunks * CHUNK
    rows_per_tile = n_pad // NS

    mesh = plsc.VectorSubcoreMesh(
        core_axis_name="c", subcore_axis_name="s", num_cores=NC,
        num_subcores=NS)

    @functools.partial(
        pl.kernel,
        out_type=(
            jax.ShapeDtypeStruct((NC, n_pad, d), jnp.float32),
            jax.ShapeDtypeStruct((NC * n_pad,), jnp.float32),
        ),
        mesh=mesh,
        scratch_types=[
            pltpu.VMEM((NBUF, CHUNK), jnp.int32),
            pltpu.VMEM((NBUF, CHUNK), jnp.int32),
            pltpu.VMEM((NBUF, CHUNK, d), jnp.float32),
            pltpu.VMEM((CHUNK,), jnp.float32),
            pltpu.VMEM((640,), jnp.float32),
            pltpu.VMEM_SHARED((n_pad, d), jnp.float32),
            pltpu.VMEM_SHARED((n_pad,), jnp.float32),
            pltpu.SemaphoreType.DMA((NBUF,)),
            pltpu.SemaphoreType.DMA((NBUF,)),
        ],
    )
    def agg(x_hbm, src_hbm, dst_hbm, zrows_hbm,
            nei_out, deg_out,
            src_ring, dst_ring, rows, ones_v, dbuf, acc, dacc, gsem, isem):
        cid = lax.axis_index("c")
        sid = lax.axis_index("s")
        wid = sid * NC + cid
        dpt = rows_per_tile

        zsl = pl.ds(sid * rows_per_tile, rows_per_tile)
        pltpu.sync_copy(zrows_hbm, acc.at[zsl])
        for j in range(640 // 16):
            dbuf[pl.ds(j * 16, 16)] = jnp.zeros((16,), jnp.float32)
        pltpu.sync_copy(dbuf.at[pl.ds(0, dpt)], dacc.at[pl.ds(sid * dpt, dpt)])
        for j in range(CHUNK // 16):
            ones_v[pl.ds(j * 16, 16)] = jnp.ones((16,), jnp.float32)
        plsc.subcore_barrier()

        def load_idx(c, slot, sem_wait=False):
            base = pl.multiple_of(wid * epw + c * CHUNK, 8)
            s = pltpu.async_copy(
                src_hbm.at[pl.ds(base, CHUNK)], src_ring.at[slot],
                isem.at[slot])
            t = pltpu.async_copy(
                dst_hbm.at[pl.ds(base, CHUNK)], dst_ring.at[slot],
                isem.at[slot])
            if sem_wait:
                s.wait()
                t.wait()

        def wait_idx(slot):
            pltpu.make_async_copy(
                src_hbm.at[pl.ds(0, CHUNK)], src_ring.at[slot],
                isem.at[slot]).wait()
            pltpu.make_async_copy(
                dst_hbm.at[pl.ds(0, CHUNK)], dst_ring.at[slot],
                isem.at[slot]).wait()

        def gather(slot):
            pltpu.async_copy(
                x_hbm.at[src_ring.at[slot]], rows.at[slot], gsem.at[slot])

        for k in range(NBUF):
            load_idx(k, k)
        for k in range(NBUF - 1):
            wait_idx(k)
            gather(k)

        def body(i, carry):
            b = lax.rem(i, NBUF)
            pltpu.make_async_copy(
                x_hbm.at[src_ring.at[0]], rows.at[b], gsem.at[b]).wait()
            pltpu.sync_copy(rows.at[b], acc.at[dst_ring.at[b]], add=True)
            pltpu.sync_copy(ones_v, dacc.at[dst_ring.at[b]], add=True)
            nl = i + NBUF

            @pl.when(nl < n_chunks)
            def _load():
                load_idx(nl, b)

            c = i + NBUF - 1
            bc = lax.rem(c, NBUF)

            @pl.when(c < n_chunks)
            def _gather():
                wait_idx(bc)
                gather(bc)
            return carry

        lax.fori_loop(0, n_chunks, body, 0)
        plsc.subcore_barrier()

        pltpu.sync_copy(acc.at[zsl], nei_out.at[cid, zsl])
        pltpu.sync_copy(dacc.at[pl.ds(sid * dpt, dpt)], dbuf.at[pl.ds(0, dpt)])
        dsl = pl.ds(pl.multiple_of(cid * n_pad + sid * dpt, 8), dpt)
        pltpu.sync_copy(dbuf.at[pl.ds(0, dpt)], deg_out.at[dsl])

    zrows = jnp.zeros((rows_per_tile, d), jnp.float32)
    return agg(x, src1, dst1, zrows)


def _linear_tc(x, nei, deg, w, b, row_block):
    n, d = x.shape

    n_pad = nei.shape[1]
    nblk = n_pad // row_block

    def body(x_ref, n_ref, d0_ref, d1_ref, w_ref, b_ref, o_ref):
        s = x_ref[...] + n_ref[0] + n_ref[1]
        dv = d0_ref[...] + d1_ref[...] + 1.0
        inv = (1.0 / dv)[:, None]
        s = s * inv
        o_ref[...] = lax.dot_general(
            s, w_ref[...], (((1,), (1,)), ((), ())),
            preferred_element_type=jnp.float32,
            precision=lax.Precision.HIGHEST) + b_ref[...]

    grid = (pl.cdiv(n, row_block),)
    return pl.pallas_call(
        body,
        grid=grid,
        in_specs=[
            pl.BlockSpec((row_block, d), lambda i: (i, 0)),
            pl.BlockSpec((NC, row_block, d), lambda i: (0, i, 0)),
            pl.BlockSpec((row_block,), lambda i: (i,)),
            pl.BlockSpec((row_block,), lambda i: (i + nblk,)),
            pl.BlockSpec((d, d), lambda i: (0, 0)),
            pl.BlockSpec((1, d), lambda i: (0, 0)),
        ],
        out_specs=pl.BlockSpec((row_block, d), lambda i: (i, 0)),
        out_shape=jax.ShapeDtypeStruct((n, d), jnp.float32),
    )(x, nei, deg, deg, w, b)


def kernel(x, edge_index, W, b):
    n, d = x.shape
    e = edge_index.shape[1]
    src = edge_index[0]
    dst = edge_index[1]

    rb = 2048
    n_pad = ((n + rb - 1) // rb) * rb
    epw = e // NW
    epw_pad = ((epw + CHUNK - 1) // CHUNK) * CHUNK
    padn = epw_pad - epw

    srcw = src.reshape(NW, epw)
    dstw = dst.reshape(NW, epw)
    if padn:
        ps = (jnp.arange(NW, dtype=jnp.int32)[:, None] * 131
              + jnp.arange(padn, dtype=jnp.int32)[None, :] * 7) % n
        pd = jnp.broadcast_to(
            n + jnp.arange(NW, dtype=jnp.int32)[:, None], (NW, padn))
        srcw = jnp.concatenate([srcw, ps], axis=1)
        dstw = jnp.concatenate([dstw, pd.astype(jnp.int32)], axis=1)
    src1 = srcw.reshape(NW * epw_pad)
    dst1 = dstw.reshape(NW * epw_pad)

    nei, deg = _aggregate_sc(x, src1, dst1, epw_pad // CHUNK, n_pad)

    return _linear_tc(x, nei, deg, W, b.reshape(1, d), row_block=rb)
